# baseline (speedup 1.0000x reference)
import functools

import jax
import jax.numpy as jnp
from jax import lax
from jax.experimental import pallas as pl
from jax.experimental.pallas import tpu as pltpu, tpu_sc as plsc

_C = 10
_L = 16
_P = 80


def _permute(v, idx):
    return lax.gather(
        v,
        idx[:, None],
        lax.GatherDimensionNumbers(
            offset_dims=(), collapsed_slice_dims=(0,), start_index_map=(0,)
        ),
        slice_sizes=(1,),
        mode=lax.GatherScatterMode.PROMISE_IN_BOUNDS,
    )


@functools.lru_cache(maxsize=None)
def _build(batch: int):
    n = batch * _C
    assert n % _P == 0
    n_workers = n // _P
    assert n_workers <= 32

    def body(w_hbm, out_hbm, w_v, buf_v):
        wid = lax.axis_index("s")

        @pl.when(wid < n_workers)
        def _():
            pltpu.sync_copy(w_hbm, w_v.at[pl.ds(0, _C)])
            wv = w_v[:]
            lane = lax.iota(jnp.int32, _L)
            valid = lane < _C
            wm = jnp.where(valid, wv, jnp.full((_L,), -jnp.inf, jnp.float32))
            wmax = wm
            for s in (8, 4, 2, 1):
                wmax = jnp.maximum(wmax, _permute(wmax, lane ^ s))
            hit = jnp.logical_and(wm == wmax, valid)
            cand = jnp.where(hit, lane, jnp.full((_L,), _L, jnp.int32))
            for s in (8, 4, 2, 1):
                cand = jnp.minimum(cand, _permute(cand, lane ^ s))
            idx = cand

            ones = jnp.ones((_L,), jnp.float32)
            zeros = jnp.zeros((_L,), jnp.float32)
            for j in range(_P // _L):
                col = lax.rem(lane + j * _L, _C)
                buf_v[pl.ds(j * _L, _L)] = jnp.where(col == idx, ones, zeros)
            pltpu.sync_copy(buf_v, out_hbm.at[pl.ds(wid * _P, _P)])

    return pl.kernel(
        body,
        mesh=plsc.VectorSubcoreMesh(
            core_axis_name="c", subcore_axis_name="s", num_cores=1
        ),
        out_type=jax.ShapeDtypeStruct((n,), jnp.float32),
        scratch_types=[
            pltpu.VMEM((_L,), jnp.float32),
            pltpu.VMEM((_P,), jnp.float32),
        ],
    )


def kernel(x, w):
    flat = _build(x.shape[0])(w.astype(jnp.float32))
    return flat.reshape(x.shape[0], _C)
